# baseline (speedup 1.0000x reference)
import functools

import jax
import jax.numpy as jnp
from jax import lax
from jax.experimental import pallas as pl
from jax.experimental.pallas import tpu as pltpu
from jax.experimental.pallas import tpu_sc as plsc

N = 10000
D = 128
E = 320000
CW = 16
NC, NS = 2, 16
EDGES_PER_TILE = E // (NC * NS)
NP = 10240
ROWS_PER_TILE = NP // NS


def _sc_aggregate(xf, pk3, with_counts, chunk, nbuf):
    CHUNK, NBUF = chunk, nbuf
    NCHUNKS = EDGES_PER_TILE // CHUNK
    ZCOPIES = ROWS_PER_TILE // CHUNK
    assert EDGES_PER_TILE % CHUNK == 0 and ROWS_PER_TILE % CHUNK == 0
    mesh = plsc.VectorSubcoreMesh(core_axis_name="c", subcore_axis_name="s")

    out_type = [jax.ShapeDtypeStruct((NC, NP, D), jnp.float32)]
    scratch = [
        pltpu.VMEM((EDGES_PER_TILE,), jnp.int32),
        pltpu.VMEM((2, NBUF, CHUNK), jnp.int32),
        pltpu.VMEM((2, NBUF, CHUNK), jnp.int32),
        pltpu.VMEM((NBUF, CHUNK, D), jnp.float32),
        pltpu.SemaphoreType.DMA((NBUF,)),
        pltpu.SemaphoreType.DMA((NBUF,)),
    ]
    if with_counts:
        out_type.append(jax.ShapeDtypeStruct((NC, NP, CW), jnp.float32))
        scratch += [
            pltpu.VMEM((CHUNK, CW), jnp.float32),
            pltpu.VMEM((CHUNK, CW), jnp.float32),
            pltpu.SemaphoreType.DMA((NBUF,)),
        ]
        scratch.append(pltpu.VMEM_SHARED((NP, CW), jnp.float32))
    scratch.append(pltpu.VMEM_SHARED((NP, D), jnp.float32))

    @functools.partial(
        pl.kernel,
        out_type=out_type,
        mesh=mesh,
        scratch_types=scratch,
        compiler_params=pltpu.CompilerParams(use_tc_tiling_on_sc=False),
    )
    def agg_kernel(xf_hbm, pk_hbm, *rest):
        if with_counts:
            (outf_hbm, outc_hbm, pkv, sidx, didx, rows, gsem, ssem,
             ones, zbuf, csem, accc, accf) = rest
        else:
            outf_hbm, pkv, sidx, didx, rows, gsem, ssem, accf = rest

        cid = lax.axis_index("c")
        sid = lax.axis_index("s")
        tid = cid * NS + sid

        pkd = pltpu.async_copy(pk_hbm.at[tid], pkv, gsem.at[0])

        @pl.loop(0, CHUNK)
        def _(i):
            @pl.loop(0, D // 16)
            def _(j):
                rows[0, i, pl.ds(j * 16, 16)] = jnp.zeros((16,), jnp.float32)
            if with_counts:
                ones[i, :] = jnp.ones((CW,), jnp.float32)
                zbuf[i, :] = jnp.zeros((CW,), jnp.float32)

        row0 = sid * ROWS_PER_TILE

        zds = []
        for r in range(ZCOPIES):
            zds.append(pltpu.async_copy(
                rows.at[0], accf.at[pl.ds(row0 + r * CHUNK, CHUNK)],
                ssem.at[0],
            ))
            if with_counts:
                zds.append(pltpu.async_copy(
                    zbuf, accc.at[pl.ds(row0 + r * CHUNK, CHUNK)], csem.at[0]
                ))
        pkd.wait()
        for d in zds:
            d.wait()

        plsc.subcore_barrier()

        ks = list(range(0, CHUNK - 15, 16))
        if CHUNK % 16:
            ks.append(CHUNK - 16)

        def unpack(g, p, b):
            off = g * CHUNK
            for k in ks:
                v = pkv[pl.ds(off + k, 16)]
                sidx[p, b, pl.ds(k, 16)] = lax.bitwise_and(v, 0xFFFF)
                didx[p, b, pl.ds(k, 16)] = lax.shift_right_logical(v, 16)

        def fire_gather(p, b):
            pltpu.async_copy(xf_hbm.at[sidx.at[p, b]], rows.at[b], gsem.at[b])

        def wait_gather(p, b):
            pltpu.make_async_copy(
                xf_hbm.at[sidx.at[p, b]], rows.at[b], gsem.at[b]
            ).wait()

        for b in range(min(NBUF, NCHUNKS)):
            unpack(b, 0, b)
            fire_gather(0, b)

        def step_body(g, q, b):
            wait_gather(q, b)
            sc = pltpu.async_copy(
                rows.at[b], accf.at[didx.at[q, b]], ssem.at[b], add=True
            )
            if with_counts:
                cc = pltpu.async_copy(
                    ones, accc.at[didx.at[q, b]], csem.at[b], add=True
                )
            g2 = g + NBUF

            @pl.when(g2 < NCHUNKS)
            def _():
                unpack(g2, 1 - q, b)

            sc.wait()
            if with_counts:
                cc.wait()

            @pl.when(g2 < NCHUNKS)
            def _():
                fire_gather(1 - q, b)

        def step(g, q, b):
            @pl.when(g < NCHUNKS)
            def _():
                step_body(g, q, b)

        @pl.loop(0, pl.cdiv(NCHUNKS, 2 * NBUF))
        def _(i):
            for q in (0, 1):
                for b in range(NBUF):
                    step(i * 2 * NBUF + q * NBUF + b, q, b)

        plsc.subcore_barrier()
        pltpu.sync_copy(
            accf.at[pl.ds(row0, ROWS_PER_TILE)],
            outf_hbm.at[cid, pl.ds(row0, ROWS_PER_TILE)],
        )
        if with_counts:
            pltpu.sync_copy(
                accc.at[pl.ds(row0, ROWS_PER_TILE)],
                outc_hbm.at[cid, pl.ds(row0, ROWS_PER_TILE)],
            )

    return agg_kernel(xf, pk3)


BN = 1000


def _tc_combine(p, co, xf, Wcat, bl):
    def body(p_ref, c_ref, x_ref, w_ref, b_ref, o_ref):
        s = p_ref[0] + p_ref[1]
        cnt = (c_ref[0] + c_ref[1])[:, 0:1]
        mean = s / jnp.maximum(cnt, 1.0)
        z = jnp.concatenate([mean, x_ref[...]], axis=1)
        out = lax.dot_general(
            z.astype(jnp.bfloat16),
            w_ref[...].astype(jnp.bfloat16),
            (((1,), (1,)), ((), ())),
            preferred_element_type=jnp.float32,
        )
        o_ref[...] = jnp.maximum(out + b_ref[...], 0.0)

    return pl.pallas_call(
        body,
        grid=(N // BN,),
        in_specs=[
            pl.BlockSpec((NC, BN, D), lambda i: (0, i, 0)),
            pl.BlockSpec((NC, BN, CW), lambda i: (0, i, 0)),
            pl.BlockSpec((BN, D), lambda i: (i, 0)),
            pl.BlockSpec((D, 2 * D), lambda i: (0, 0)),
            pl.BlockSpec((1, D), lambda i: (0, 0)),
        ],
        out_specs=pl.BlockSpec((BN, D), lambda i: (i, 0)),
        out_shape=jax.ShapeDtypeStruct((N, D), jnp.float32),
    )(p, co, xf, Wcat, bl.reshape(1, D))


def kernel(x, edge_index, W_l1, b_l1, W_r1, W_l2, b_l2, W_r2):
    src = edge_index[0]
    dst = edge_index[1]
    pk3 = (src | (dst << 16)).reshape(NC * NS, EDGES_PER_TILE)

    Wcat1 = jnp.concatenate([W_l1, W_r1], axis=1)
    Wcat2 = jnp.concatenate([W_l2, W_r2], axis=1)

    p1, c1 = _sc_aggregate(x, pk3, with_counts=True, chunk=40, nbuf=5)
    h = _tc_combine(p1, c1, x, Wcat1, b_l1)

    (p2,) = _sc_aggregate(h, pk3, with_counts=False, chunk=40, nbuf=5)
    return _tc_combine(p2, c1, h, Wcat2, b_l2)


# speedup vs baseline: 1.0017x; 1.0017x over previous
import functools

import jax
import jax.numpy as jnp
from jax import lax
from jax.experimental import pallas as pl
from jax.experimental.pallas import tpu as pltpu
from jax.experimental.pallas import tpu_sc as plsc

N = 10000
D = 128
E = 320000
CW = 16
NC, NS = 2, 16
EDGES_PER_TILE = E // (NC * NS)
NP = 10240
ROWS_PER_TILE = NP // NS


def _sc_aggregate(xf, pk3, with_counts, chunk, nbuf):
    CHUNK, NBUF = chunk, nbuf
    NCHUNKS = EDGES_PER_TILE // CHUNK
    ZCOPIES = ROWS_PER_TILE // CHUNK
    assert EDGES_PER_TILE % CHUNK == 0 and ROWS_PER_TILE % CHUNK == 0
    mesh = plsc.VectorSubcoreMesh(core_axis_name="c", subcore_axis_name="s")

    out_type = [jax.ShapeDtypeStruct((NC, NP, D), jnp.float32)]
    scratch = [
        pltpu.VMEM((EDGES_PER_TILE,), jnp.int32),
        pltpu.VMEM((2, NBUF, CHUNK), jnp.int32),
        pltpu.VMEM((2, NBUF, CHUNK), jnp.int32),
        pltpu.VMEM((NBUF, CHUNK, D), jnp.float32),
        pltpu.SemaphoreType.DMA((NBUF,)),
        pltpu.SemaphoreType.DMA((NBUF,)),
    ]
    if with_counts:
        out_type.append(jax.ShapeDtypeStruct((NC, NP, CW), jnp.float32))
        scratch += [
            pltpu.VMEM((CHUNK, CW), jnp.float32),
            pltpu.VMEM((CHUNK, CW), jnp.float32),
            pltpu.SemaphoreType.DMA((NBUF,)),
        ]
        scratch.append(pltpu.VMEM_SHARED((NP, CW), jnp.float32))
    scratch.append(pltpu.VMEM_SHARED((NP, D), jnp.float32))

    @functools.partial(
        pl.kernel,
        out_type=out_type,
        mesh=mesh,
        scratch_types=scratch,
        compiler_params=pltpu.CompilerParams(use_tc_tiling_on_sc=False),
    )
    def agg_kernel(xf_hbm, pk_hbm, *rest):
        if with_counts:
            (outf_hbm, outc_hbm, pkv, sidx, didx, rows, gsem, ssem,
             ones, zbuf, csem, accc, accf) = rest
        else:
            outf_hbm, pkv, sidx, didx, rows, gsem, ssem, accf = rest

        cid = lax.axis_index("c")
        sid = lax.axis_index("s")
        tid = cid * NS + sid

        pkd = pltpu.async_copy(pk_hbm.at[tid], pkv, gsem.at[0])

        @pl.loop(0, CHUNK)
        def _(i):
            @pl.loop(0, D // 16)
            def _(j):
                rows[0, i, pl.ds(j * 16, 16)] = jnp.zeros((16,), jnp.float32)
            if with_counts:
                ones[i, :] = jnp.ones((CW,), jnp.float32)
                zbuf[i, :] = jnp.zeros((CW,), jnp.float32)

        row0 = sid * ROWS_PER_TILE

        zds = []
        for r in range(ZCOPIES):
            zds.append(pltpu.async_copy(
                rows.at[0], accf.at[pl.ds(row0 + r * CHUNK, CHUNK)],
                ssem.at[0],
            ))
            if with_counts:
                zds.append(pltpu.async_copy(
                    zbuf, accc.at[pl.ds(row0 + r * CHUNK, CHUNK)], csem.at[0]
                ))
        pkd.wait()
        for d in zds:
            d.wait()

        plsc.subcore_barrier()

        ks = list(range(0, CHUNK - 15, 16))
        if CHUNK % 16:
            ks.append(CHUNK - 16)

        def unpack(g, p, b):
            off = g * CHUNK
            for k in ks:
                v = pkv[pl.ds(off + k, 16)]
                sidx[p, b, pl.ds(k, 16)] = lax.bitwise_and(v, 0xFFFF)
                didx[p, b, pl.ds(k, 16)] = lax.shift_right_logical(v, 16)

        def fire_gather(p, b):
            pltpu.async_copy(xf_hbm.at[sidx.at[p, b]], rows.at[b], gsem.at[b])

        def wait_gather(p, b):
            pltpu.make_async_copy(
                xf_hbm.at[sidx.at[p, b]], rows.at[b], gsem.at[b]
            ).wait()

        for b in range(min(NBUF, NCHUNKS)):
            unpack(b, 0, b)
            fire_gather(0, b)

        def step_body(g, q, b):
            wait_gather(q, b)
            sc = pltpu.async_copy(
                rows.at[b], accf.at[didx.at[q, b]], ssem.at[b], add=True
            )
            if with_counts:
                cc = pltpu.async_copy(
                    ones, accc.at[didx.at[q, b]], csem.at[b], add=True
                )
            g2 = g + NBUF

            @pl.when(g2 < NCHUNKS)
            def _():
                unpack(g2, 1 - q, b)

            sc.wait()
            if with_counts:
                cc.wait()

            @pl.when(g2 < NCHUNKS)
            def _():
                fire_gather(1 - q, b)

        def step(g, q, b):
            @pl.when(g < NCHUNKS)
            def _():
                step_body(g, q, b)

        @pl.loop(0, pl.cdiv(NCHUNKS, 2 * NBUF))
        def _(i):
            for q in (0, 1):
                for b in range(NBUF):
                    step(i * 2 * NBUF + q * NBUF + b, q, b)

        plsc.subcore_barrier()
        pltpu.sync_copy(
            accf.at[pl.ds(row0, ROWS_PER_TILE)],
            outf_hbm.at[cid, pl.ds(row0, ROWS_PER_TILE)],
        )
        if with_counts:
            pltpu.sync_copy(
                accc.at[pl.ds(row0, ROWS_PER_TILE)],
                outc_hbm.at[cid, pl.ds(row0, ROWS_PER_TILE)],
            )

    return agg_kernel(xf, pk3)


BN = 1000


def _tc_combine(p, co, xf, Wcat, bl):
    def body(p_ref, c_ref, x_ref, w_ref, b_ref, o_ref):
        s = p_ref[0] + p_ref[1]
        cnt = (c_ref[0] + c_ref[1])[:, 0:1]
        mean = s / jnp.maximum(cnt, 1.0)
        z = jnp.concatenate([mean, x_ref[...]], axis=1)
        out = lax.dot_general(
            z,
            w_ref[...],
            (((1,), (1,)), ((), ())),
            preferred_element_type=jnp.float32,
        )
        o_ref[...] = jnp.maximum(out + b_ref[...], 0.0)

    return pl.pallas_call(
        body,
        grid=(N // BN,),
        in_specs=[
            pl.BlockSpec((NC, BN, D), lambda i: (0, i, 0)),
            pl.BlockSpec((NC, BN, CW), lambda i: (0, i, 0)),
            pl.BlockSpec((BN, D), lambda i: (i, 0)),
            pl.BlockSpec((D, 2 * D), lambda i: (0, 0)),
            pl.BlockSpec((1, D), lambda i: (0, 0)),
        ],
        out_specs=pl.BlockSpec((BN, D), lambda i: (i, 0)),
        out_shape=jax.ShapeDtypeStruct((N, D), jnp.float32),
    )(p, co, xf, Wcat, bl.reshape(1, D))


def kernel(x, edge_index, W_l1, b_l1, W_r1, W_l2, b_l2, W_r2):
    src = edge_index[0]
    dst = edge_index[1]
    pk3 = (src | (dst << 16)).reshape(NC * NS, EDGES_PER_TILE)

    Wcat1 = jnp.concatenate([W_l1, W_r1], axis=1)
    Wcat2 = jnp.concatenate([W_l2, W_r2], axis=1)

    p1, c1 = _sc_aggregate(x, pk3, with_counts=True, chunk=40, nbuf=5)
    h = _tc_combine(p1, c1, x, Wcat1, b_l1)

    (p2,) = _sc_aggregate(h, pk3, with_counts=False, chunk=40, nbuf=5)
    return _tc_combine(p2, c1, h, Wcat2, b_l2)
